# baseline (speedup 1.0000x reference)
import functools

import jax
import jax.numpy as jnp
from jax import lax
from jax.experimental import pallas as pl
from jax.experimental.pallas import tpu as pltpu
from jax.experimental.pallas import tpu_sc as plsc

N = 10000
D = 128
E = 320000
NP = 10240
C = 80
NSC = 2
NSUB = 16
NW = NSC * NSUB
K = 125
EPW = K * C
RPT = NP // NSUB
KM = 123

_sc_mesh = plsc.VectorSubcoreMesh(core_axis_name="c", subcore_axis_name="s")


@functools.partial(
    pl.kernel,
    out_type=jax.ShapeDtypeStruct((NSC, NP), jnp.float32),
    mesh=_sc_mesh,
    scratch_types=[
        pltpu.VMEM((K, C), jnp.int32),
        pltpu.VMEM((C,), jnp.float32),
        pltpu.VMEM((RPT,), jnp.float32),
        pltpu.VMEM_SHARED((NP,), jnp.float32),
        pltpu.SemaphoreType.DMA,
    ],
)
def _deg_kernel(dst_hbm, out_hbm, didx_v, ones_v, zrow_v, deg_sh, sem):
    cid = lax.axis_index("c")
    sid = lax.axis_index("s")
    wid = sid * NSC + cid
    for i in range(C // 16):
        ones_v[pl.ds(i * 16, 16)] = jnp.ones((16,), jnp.float32)
    for i in range(RPT // 16):
        zrow_v[pl.ds(i * 16, 16)] = jnp.zeros((16,), jnp.float32)
    r0 = sid * RPT
    pltpu.sync_copy(zrow_v, deg_sh.at[pl.ds(r0, RPT)])
    pltpu.sync_copy(dst_hbm.at[wid], didx_v)
    plsc.subcore_barrier()

    @pl.loop(0, K)
    def _fire(j):
        pltpu.async_copy(ones_v, deg_sh.at[didx_v.at[j]], sem, add=True)

    @pl.loop(0, K)
    def _drain(j):
        pltpu.make_async_copy(ones_v, deg_sh.at[didx_v.at[0]], sem).wait()

    plsc.subcore_barrier()
    pltpu.sync_copy(deg_sh.at[pl.ds(r0, RPT)], out_hbm.at[cid, pl.ds(r0, RPT)])


@functools.partial(
    pl.kernel,
    out_type=jax.ShapeDtypeStruct((NSC, NP, D), jnp.float32),
    mesh=_sc_mesh,
    scratch_types=[
        pltpu.VMEM((EPW,), jnp.int32),
        [pltpu.VMEM((1, C), jnp.int32) for _ in range(3)],
        [pltpu.VMEM((C, D), jnp.float32) for _ in range(3)],
        pltpu.VMEM_SHARED((NP, D), jnp.float32),
        [pltpu.SemaphoreType.DMA for _ in range(3)],
        [pltpu.SemaphoreType.DMA for _ in range(3)],
        [pltpu.SemaphoreType.DMA for _ in range(3)],
    ],
)
def _agg_kernel(g_hbm, src_hbm, dst_hbm, zero_hbm, out_hbm,
                sidx_v, didx, rows, acc_sh, gsem, ssem, isem):
    cid = lax.axis_index("c")
    sid = lax.axis_index("s")
    wid = sid * NSC + cid
    r0 = sid * RPT
    pltpu.sync_copy(zero_hbm.at[pl.ds(r0, RPT)], acc_sh.at[pl.ds(r0, RPT)])
    pltpu.sync_copy(src_hbm.at[wid], sidx_v)
    plsc.subcore_barrier()

    def iload(j, b):
        pltpu.async_copy(dst_hbm.at[wid, j], didx[b].at[0], isem[b])

    def iwait(b):
        pltpu.make_async_copy(dst_hbm.at[wid, 0], didx[b].at[0],
                              isem[b]).wait()

    def gather(j, b):
        pltpu.async_copy(g_hbm.at[sidx_v.at[pl.ds(j * C, C)]], rows[b],
                         gsem[b])

    def gwait(b):
        pltpu.make_async_copy(g_hbm.at[sidx_v.at[pl.ds(0, C)]], rows[b],
                              gsem[b]).wait()

    def scat(b):
        pltpu.async_copy(rows[b], acc_sh.at[didx[b].at[0]], ssem[b], add=True)

    def swait(b):
        pltpu.make_async_copy(rows[b], acc_sh.at[didx[b].at[0]],
                              ssem[b]).wait()

    for b in range(3):
        iload(b, b)
        gather(b, b)

    @pl.loop(0, KM, step=3)
    def _edges(j):
        for b in range(3):
            gwait(b)
            iwait(b)
            scat(b)
            swait(b)

            @pl.when(j + b + 3 < K)
            def _():
                iload(j + b + 3, b)
                gather(j + b + 3, b)

    for b in range(K - KM):
        gwait(b)
        iwait(b)
        scat(b)
        swait(b)

    plsc.subcore_barrier()
    pltpu.sync_copy(acc_sh.at[pl.ds(r0, RPT)],
                    out_hbm.at[cid, pl.ds(r0, RPT)])


_BM = 1000


def _g_body(x_ref, w_ref, pt_ref, g_ref):
    d = pt_ref[:, 0] + pt_ref[:, 1] + 1.0
    dinv = lax.rsqrt(d)
    h = jnp.dot(x_ref[:, :], w_ref[:, :], preferred_element_type=jnp.float32,
                precision="highest")
    g_ref[:, :] = h * dinv[:, None]


_g_call = pl.pallas_call(
    _g_body,
    grid=(N // _BM,),
    in_specs=[
        pl.BlockSpec((_BM, D), lambda i: (i, 0)),
        pl.BlockSpec((D, D), lambda i: (0, 0)),
        pl.BlockSpec((_BM, 2), lambda i: (i, 0)),
    ],
    out_specs=pl.BlockSpec((_BM, D), lambda i: (i, 0)),
    out_shape=jax.ShapeDtypeStruct((N, D), jnp.float32),
)


def _fin_body(acc_ref, g_ref, pt_ref, b_ref, o_ref):
    d = pt_ref[:, 0] + pt_ref[:, 1] + 1.0
    dinv = lax.rsqrt(d)
    s = acc_ref[0] + acc_ref[1] + g_ref[:, :]
    o_ref[:, :] = s * dinv[:, None] + b_ref[0]


_fin_call = pl.pallas_call(
    _fin_body,
    grid=(N // _BM,),
    in_specs=[
        pl.BlockSpec((NSC, _BM, D), lambda i: (0, i, 0)),
        pl.BlockSpec((_BM, D), lambda i: (i, 0)),
        pl.BlockSpec((_BM, 2), lambda i: (i, 0)),
        pl.BlockSpec((1, D), lambda i: (0, 0)),
    ],
    out_specs=pl.BlockSpec((_BM, D), lambda i: (i, 0)),
    out_shape=jax.ShapeDtypeStruct((N, D), jnp.float32),
)


def kernel(x, edge_index, W, b):
    src_p = edge_index[0].reshape(NW, EPW)
    dst_p = edge_index[1].reshape(NW, K, C)
    degp = _deg_kernel(dst_p)
    pt = degp.T
    g = _g_call(x, W, pt)
    zeros_nd = jnp.zeros((NP, D), jnp.float32)
    accs = _agg_kernel(g, src_p, dst_p, zeros_nd)
    out = _fin_call(accs, g, pt, b.reshape(1, D))
    return out


# speedup vs baseline: 1.0408x; 1.0408x over previous
import functools

import jax
import jax.numpy as jnp
from jax import lax
from jax.experimental import pallas as pl
from jax.experimental.pallas import tpu as pltpu
from jax.experimental.pallas import tpu_sc as plsc

N = 10000
D = 128
E = 320000
NP = 10240
C = 80
NSC = 2
NSUB = 16
NW = NSC * NSUB
K = 125
EPW = K * C
RPT = NP // NSUB
KM = 123

_sc_mesh = plsc.VectorSubcoreMesh(core_axis_name="c", subcore_axis_name="s")


@functools.partial(
    pl.kernel,
    out_type=jax.ShapeDtypeStruct((NSC, NP), jnp.float32),
    mesh=_sc_mesh,
    scratch_types=[
        pltpu.VMEM((K, C), jnp.int32),
        pltpu.VMEM((C,), jnp.float32),
        pltpu.VMEM((RPT,), jnp.float32),
        pltpu.VMEM_SHARED((NP,), jnp.float32),
        pltpu.SemaphoreType.DMA,
    ],
)
def _deg_kernel(dst_hbm, out_hbm, didx_v, ones_v, zrow_v, deg_sh, sem):
    cid = lax.axis_index("c")
    sid = lax.axis_index("s")
    wid = sid * NSC + cid
    for i in range(C // 16):
        ones_v[pl.ds(i * 16, 16)] = jnp.ones((16,), jnp.float32)
    for i in range(RPT // 16):
        zrow_v[pl.ds(i * 16, 16)] = jnp.zeros((16,), jnp.float32)
    r0 = sid * RPT
    pltpu.sync_copy(zrow_v, deg_sh.at[pl.ds(r0, RPT)])
    pltpu.sync_copy(dst_hbm.at[wid], didx_v)
    plsc.subcore_barrier()

    @pl.loop(0, K)
    def _fire(j):
        pltpu.async_copy(ones_v, deg_sh.at[didx_v.at[j]], sem, add=True)

    @pl.loop(0, K)
    def _drain(j):
        pltpu.make_async_copy(ones_v, deg_sh.at[didx_v.at[0]], sem).wait()

    plsc.subcore_barrier()
    pltpu.sync_copy(deg_sh.at[pl.ds(r0, RPT)], out_hbm.at[cid, pl.ds(r0, RPT)])


@functools.partial(
    pl.kernel,
    out_type=jax.ShapeDtypeStruct((NSC, NP, D), jnp.float32),
    mesh=_sc_mesh,
    scratch_types=[
        pltpu.VMEM((EPW,), jnp.int32),
        [pltpu.VMEM((1, C), jnp.int32) for _ in range(3)],
        [pltpu.VMEM((C, D), jnp.float32) for _ in range(3)],
        pltpu.VMEM((16, D), jnp.float32),
        pltpu.VMEM_SHARED((NP, D), jnp.float32),
        [pltpu.SemaphoreType.DMA for _ in range(3)],
        [pltpu.SemaphoreType.DMA for _ in range(3)],
        [pltpu.SemaphoreType.DMA for _ in range(3)],
        pltpu.SemaphoreType.DMA,
    ],
)
def _agg_kernel(g_hbm, src_hbm, dst_hbm, out_hbm,
                sidx_v, didx, rows, zbuf, acc_sh, gsem, ssem, isem, zsem):
    cid = lax.axis_index("c")
    sid = lax.axis_index("s")
    wid = sid * NSC + cid
    r0 = sid * RPT
    pltpu.sync_copy(src_hbm.at[wid], sidx_v)

    def iload(j, b):
        pltpu.async_copy(dst_hbm.at[wid, j], didx[b].at[0], isem[b])

    def iwait(b):
        pltpu.make_async_copy(dst_hbm.at[wid, 0], didx[b].at[0],
                              isem[b]).wait()

    def gather(j, b):
        pltpu.async_copy(g_hbm.at[sidx_v.at[pl.ds(j * C, C)]], rows[b],
                         gsem[b])

    def gwait(b):
        pltpu.make_async_copy(g_hbm.at[sidx_v.at[pl.ds(0, C)]], rows[b],
                              gsem[b]).wait()

    def scat(b):
        pltpu.async_copy(rows[b], acc_sh.at[didx[b].at[0]], ssem[b], add=True)

    def swait(b):
        pltpu.make_async_copy(rows[b], acc_sh.at[didx[b].at[0]],
                              ssem[b]).wait()

    for b in range(3):
        iload(b, b)
        gather(b, b)

    for i in range(16):
        for k in range(D // 16):
            zbuf[i, pl.ds(k * 16, 16)] = jnp.zeros((16,), jnp.float32)
    for i in range(RPT // 16):
        pltpu.async_copy(zbuf, acc_sh.at[pl.ds(r0 + i * 16, 16)], zsem)
    for i in range(RPT // 16):
        pltpu.make_async_copy(zbuf, acc_sh.at[pl.ds(r0, 16)], zsem).wait()
    plsc.subcore_barrier()

    @pl.loop(0, KM, step=3)
    def _edges(j):
        for b in range(3):
            gwait(b)
            iwait(b)
            scat(b)
            swait(b)

            @pl.when(j + b + 3 < K)
            def _():
                iload(j + b + 3, b)
                gather(j + b + 3, b)

    for b in range(K - KM):
        gwait(b)
        iwait(b)
        scat(b)
        swait(b)

    plsc.subcore_barrier()
    pltpu.sync_copy(acc_sh.at[pl.ds(r0, RPT)],
                    out_hbm.at[cid, pl.ds(r0, RPT)])


_BM = 1000


def _g_body(x_ref, w_ref, pt_ref, g_ref):
    d = pt_ref[:, 0] + pt_ref[:, 1] + 1.0
    dinv = lax.rsqrt(d)
    h = jnp.dot(x_ref[:, :], w_ref[:, :], preferred_element_type=jnp.float32,
                precision="highest")
    g_ref[:, :] = h * dinv[:, None]


_g_call = pl.pallas_call(
    _g_body,
    grid=(N // _BM,),
    in_specs=[
        pl.BlockSpec((_BM, D), lambda i: (i, 0)),
        pl.BlockSpec((D, D), lambda i: (0, 0)),
        pl.BlockSpec((_BM, 2), lambda i: (i, 0)),
    ],
    out_specs=pl.BlockSpec((_BM, D), lambda i: (i, 0)),
    out_shape=jax.ShapeDtypeStruct((N, D), jnp.float32),
)


def _fin_body(acc_ref, g_ref, pt_ref, b_ref, o_ref):
    d = pt_ref[:, 0] + pt_ref[:, 1] + 1.0
    dinv = lax.rsqrt(d)
    s = acc_ref[0] + acc_ref[1] + g_ref[:, :]
    o_ref[:, :] = s * dinv[:, None] + b_ref[0]


_fin_call = pl.pallas_call(
    _fin_body,
    grid=(N // _BM,),
    in_specs=[
        pl.BlockSpec((NSC, _BM, D), lambda i: (0, i, 0)),
        pl.BlockSpec((_BM, D), lambda i: (i, 0)),
        pl.BlockSpec((_BM, 2), lambda i: (i, 0)),
        pl.BlockSpec((1, D), lambda i: (0, 0)),
    ],
    out_specs=pl.BlockSpec((_BM, D), lambda i: (i, 0)),
    out_shape=jax.ShapeDtypeStruct((N, D), jnp.float32),
)


def kernel(x, edge_index, W, b):
    src_p = edge_index[0].reshape(NW, EPW)
    dst_p = edge_index[1].reshape(NW, K, C)
    degp = _deg_kernel(dst_p)
    pt = degp.T
    g = _g_call(x, W, pt)
    accs = _agg_kernel(g, src_p, dst_p)
    out = _fin_call(accs, g, pt, b.reshape(1, D))
    return out


# speedup vs baseline: 1.1545x; 1.1093x over previous
import functools

import jax
import jax.numpy as jnp
from jax import lax
from jax.experimental import pallas as pl
from jax.experimental.pallas import tpu as pltpu
from jax.experimental.pallas import tpu_sc as plsc

N = 10000
D = 128
E = 320000
NP = 10240
C = 80
NSC = 2
NSUB = 16
NW = NSC * NSUB
K = 125
EPW = K * C
RPT = NP // NSUB
KM = 123

_sc_mesh = plsc.VectorSubcoreMesh(core_axis_name="c", subcore_axis_name="s")


@functools.partial(
    pl.kernel,
    out_type=jax.ShapeDtypeStruct((NSC, NP), jnp.float32),
    mesh=_sc_mesh,
    scratch_types=[
        pltpu.VMEM((K, C), jnp.int32),
        pltpu.VMEM((C,), jnp.float32),
        pltpu.VMEM((RPT,), jnp.float32),
        pltpu.VMEM_SHARED((NP,), jnp.float32),
        pltpu.SemaphoreType.DMA,
        pltpu.SemaphoreType.DMA,
    ],
)
def _deg_kernel(edge_hbm, out_hbm, didx_v, ones_v, zrow_v, deg_sh, sem, isem):
    cid = lax.axis_index("c")
    sid = lax.axis_index("s")
    wid = sid * NSC + cid
    base = wid * EPW

    @pl.loop(0, K)
    def _il(j):
        pltpu.async_copy(edge_hbm.at[pl.ds(E + base + j * C, C)],
                         didx_v.at[j], isem)

    for i in range(C // 16):
        ones_v[pl.ds(i * 16, 16)] = jnp.ones((16,), jnp.float32)
    for i in range(RPT // 16):
        zrow_v[pl.ds(i * 16, 16)] = jnp.zeros((16,), jnp.float32)
    r0 = sid * RPT
    pltpu.sync_copy(zrow_v, deg_sh.at[pl.ds(r0, RPT)])

    @pl.loop(0, K)
    def _ilw(j):
        pltpu.make_async_copy(edge_hbm.at[pl.ds(E + base, C)],
                              didx_v.at[0], isem).wait()

    plsc.subcore_barrier()

    @pl.loop(0, K)
    def _fire(j):
        pltpu.async_copy(ones_v, deg_sh.at[didx_v.at[j]], sem, add=True)

    @pl.loop(0, K)
    def _drain(j):
        pltpu.make_async_copy(ones_v, deg_sh.at[didx_v.at[0]], sem).wait()

    plsc.subcore_barrier()
    pltpu.sync_copy(deg_sh.at[pl.ds(r0, RPT)], out_hbm.at[cid, pl.ds(r0, RPT)])


@functools.partial(
    pl.kernel,
    out_type=jax.ShapeDtypeStruct((NSC, NP, D), jnp.float32),
    mesh=_sc_mesh,
    scratch_types=[
        pltpu.VMEM((EPW,), jnp.int32),
        [pltpu.VMEM((1, C), jnp.int32) for _ in range(3)],
        [pltpu.VMEM((C, D), jnp.float32) for _ in range(3)],
        pltpu.VMEM((16, D), jnp.float32),
        pltpu.VMEM_SHARED((NP, D), jnp.float32),
        [pltpu.SemaphoreType.DMA for _ in range(3)],
        [pltpu.SemaphoreType.DMA for _ in range(3)],
        [pltpu.SemaphoreType.DMA for _ in range(3)],
        pltpu.SemaphoreType.DMA,
    ],
)
def _agg_kernel(g_hbm, edge_hbm, out_hbm,
                sidx_v, didx, rows, zbuf, acc_sh, gsem, ssem, isem, zsem):
    cid = lax.axis_index("c")
    sid = lax.axis_index("s")
    wid = sid * NSC + cid
    r0 = sid * RPT
    base = wid * EPW
    pltpu.sync_copy(edge_hbm.at[pl.ds(base, EPW)], sidx_v)

    def iload(j, b):
        pltpu.async_copy(edge_hbm.at[pl.ds(E + base + j * C, C)],
                         didx[b].at[0], isem[b])

    def iwait(b):
        pltpu.make_async_copy(edge_hbm.at[pl.ds(E + base, C)],
                              didx[b].at[0], isem[b]).wait()

    def gather(j, b):
        pltpu.async_copy(g_hbm.at[sidx_v.at[pl.ds(j * C, C)]], rows[b],
                         gsem[b])

    def gwait(b):
        pltpu.make_async_copy(g_hbm.at[sidx_v.at[pl.ds(0, C)]], rows[b],
                              gsem[b]).wait()

    def scat(b):
        pltpu.async_copy(rows[b], acc_sh.at[didx[b].at[0]], ssem[b], add=True)

    def swait(b):
        pltpu.make_async_copy(rows[b], acc_sh.at[didx[b].at[0]],
                              ssem[b]).wait()

    for b in range(3):
        iload(b, b)
        gather(b, b)

    for i in range(16):
        for k in range(D // 16):
            zbuf[i, pl.ds(k * 16, 16)] = jnp.zeros((16,), jnp.float32)
    for i in range(RPT // 16):
        pltpu.async_copy(zbuf, acc_sh.at[pl.ds(r0 + i * 16, 16)], zsem)
    for i in range(RPT // 16):
        pltpu.make_async_copy(zbuf, acc_sh.at[pl.ds(r0, 16)], zsem).wait()
    plsc.subcore_barrier()

    @pl.loop(0, KM, step=3)
    def _edges(j):
        for b in range(3):
            gwait(b)
            iwait(b)
            scat(b)
            swait(b)

            @pl.when(j + b + 3 < K)
            def _():
                iload(j + b + 3, b)
                gather(j + b + 3, b)

    for b in range(K - KM):
        gwait(b)
        iwait(b)
        scat(b)
        swait(b)

    plsc.subcore_barrier()
    pltpu.sync_copy(acc_sh.at[pl.ds(r0, RPT)],
                    out_hbm.at[cid, pl.ds(r0, RPT)])


_BM = 1000


def _g_body(x_ref, w_ref, pt_ref, g_ref):
    d = pt_ref[:, 0] + pt_ref[:, 1] + 1.0
    dinv = lax.rsqrt(d)
    h = jnp.dot(x_ref[:, :], w_ref[:, :], preferred_element_type=jnp.float32)
    g_ref[:, :] = h * dinv[:, None]


_g_call = pl.pallas_call(
    _g_body,
    grid=(N // _BM,),
    in_specs=[
        pl.BlockSpec((_BM, D), lambda i: (i, 0)),
        pl.BlockSpec((D, D), lambda i: (0, 0)),
        pl.BlockSpec((_BM, 2), lambda i: (i, 0)),
    ],
    out_specs=pl.BlockSpec((_BM, D), lambda i: (i, 0)),
    out_shape=jax.ShapeDtypeStruct((N, D), jnp.float32),
)


def _fin_body(acc_ref, g_ref, pt_ref, b_ref, o_ref):
    d = pt_ref[:, 0] + pt_ref[:, 1] + 1.0
    dinv = lax.rsqrt(d)
    s = acc_ref[0] + acc_ref[1] + g_ref[:, :]
    o_ref[:, :] = s * dinv[:, None] + b_ref[0]


_fin_call = pl.pallas_call(
    _fin_body,
    grid=(N // _BM,),
    in_specs=[
        pl.BlockSpec((NSC, _BM, D), lambda i: (0, i, 0)),
        pl.BlockSpec((_BM, D), lambda i: (i, 0)),
        pl.BlockSpec((_BM, 2), lambda i: (i, 0)),
        pl.BlockSpec((1, D), lambda i: (0, 0)),
    ],
    out_specs=pl.BlockSpec((_BM, D), lambda i: (i, 0)),
    out_shape=jax.ShapeDtypeStruct((N, D), jnp.float32),
)


def kernel(x, edge_index, W, b):
    e_flat = edge_index.reshape(2 * E)
    degp = _deg_kernel(e_flat)
    pt = degp.T
    g = _g_call(x, W, pt)
    accs = _agg_kernel(g, e_flat)
    out = _fin_call(accs, g, pt, b.reshape(1, D))
    return out


# speedup vs baseline: 1.1916x; 1.0321x over previous
import functools

import jax
import jax.numpy as jnp
from jax import lax
from jax.experimental import pallas as pl
from jax.experimental.pallas import tpu as pltpu
from jax.experimental.pallas import tpu_sc as plsc

N = 10000
D = 128
E = 320000
NP = 10240
C = 80
NSC = 2
NSUB = 16
NW = NSC * NSUB
K = 125
EPW = K * C
RPT = NP // NSUB
KM = 120

_sc_mesh = plsc.VectorSubcoreMesh(core_axis_name="c", subcore_axis_name="s")


@functools.partial(
    pl.kernel,
    out_type=jax.ShapeDtypeStruct((NSC, NP), jnp.float32),
    mesh=_sc_mesh,
    scratch_types=[
        pltpu.VMEM((K, C), jnp.int32),
        pltpu.VMEM((C,), jnp.float32),
        pltpu.VMEM((RPT,), jnp.float32),
        pltpu.VMEM_SHARED((NP,), jnp.float32),
        pltpu.SemaphoreType.DMA,
        pltpu.SemaphoreType.DMA,
    ],
)
def _deg_kernel(edge_hbm, out_hbm, didx_v, ones_v, zrow_v, deg_sh, sem, isem):
    cid = lax.axis_index("c")
    sid = lax.axis_index("s")
    wid = sid * NSC + cid
    base = wid * EPW

    @pl.loop(0, K)
    def _il(j):
        pltpu.async_copy(edge_hbm.at[pl.ds(E + base + j * C, C)],
                         didx_v.at[j], isem)

    for i in range(C // 16):
        ones_v[pl.ds(i * 16, 16)] = jnp.ones((16,), jnp.float32)
    for i in range(RPT // 16):
        zrow_v[pl.ds(i * 16, 16)] = jnp.zeros((16,), jnp.float32)
    r0 = sid * RPT
    pltpu.sync_copy(zrow_v, deg_sh.at[pl.ds(r0, RPT)])

    @pl.loop(0, K)
    def _ilw(j):
        pltpu.make_async_copy(edge_hbm.at[pl.ds(E + base, C)],
                              didx_v.at[0], isem).wait()

    plsc.subcore_barrier()

    @pl.loop(0, K)
    def _fire(j):
        pltpu.async_copy(ones_v, deg_sh.at[didx_v.at[j]], sem, add=True)

    @pl.loop(0, K)
    def _drain(j):
        pltpu.make_async_copy(ones_v, deg_sh.at[didx_v.at[0]], sem).wait()

    plsc.subcore_barrier()
    pltpu.sync_copy(deg_sh.at[pl.ds(r0, RPT)], out_hbm.at[cid, pl.ds(r0, RPT)])


@functools.partial(
    pl.kernel,
    out_type=jax.ShapeDtypeStruct((NSC, NP, D), jnp.float32),
    mesh=_sc_mesh,
    scratch_types=[
        [pltpu.VMEM((1, C), jnp.int32) for _ in range(8)],
        [pltpu.VMEM((1, C), jnp.int32) for _ in range(8)],
        [pltpu.VMEM((C, D), jnp.float32) for _ in range(4)],
        pltpu.VMEM((16, D), jnp.float32),
        pltpu.VMEM_SHARED((NP, D), jnp.float32),
        [pltpu.SemaphoreType.DMA for _ in range(8)],
        [pltpu.SemaphoreType.DMA for _ in range(4)],
        [pltpu.SemaphoreType.DMA for _ in range(4)],
        pltpu.SemaphoreType.DMA,
    ],
)
def _agg_kernel(g_hbm, edge_hbm, out_hbm,
                sidx, didx, rows, zbuf, acc_sh, isem, gsem, ssem, zsem):
    cid = lax.axis_index("c")
    sid = lax.axis_index("s")
    wid = sid * NSC + cid
    r0 = sid * RPT
    base = wid * EPW

    def iload(j, bi):
        pltpu.async_copy(edge_hbm.at[pl.ds(base + j * C, C)],
                         sidx[bi].at[0], isem[bi])
        pltpu.async_copy(edge_hbm.at[pl.ds(E + base + j * C, C)],
                         didx[bi].at[0], isem[bi])

    def iwait(bi):
        pltpu.make_async_copy(edge_hbm.at[pl.ds(base, C)],
                              sidx[bi].at[0], isem[bi]).wait()
        pltpu.make_async_copy(edge_hbm.at[pl.ds(E + base, C)],
                              didx[bi].at[0], isem[bi]).wait()

    def gather(bi, br):
        pltpu.async_copy(g_hbm.at[sidx[bi].at[0]], rows[br], gsem[br])

    def gwait(br):
        pltpu.make_async_copy(g_hbm.at[sidx[0].at[0]], rows[br],
                              gsem[br]).wait()

    def scat(bi, br):
        pltpu.async_copy(rows[br], acc_sh.at[didx[bi].at[0]], ssem[br],
                         add=True)

    def swait(br):
        pltpu.make_async_copy(rows[br], acc_sh.at[didx[0].at[0]],
                              ssem[br]).wait()

    for j in range(8):
        iload(j, j)
    for b in range(4):
        iwait(b)
        gather(b, b)

    for i in range(16):
        for k in range(D // 16):
            zbuf[i, pl.ds(k * 16, 16)] = jnp.zeros((16,), jnp.float32)
    for i in range(RPT // 16):
        pltpu.async_copy(zbuf, acc_sh.at[pl.ds(r0 + i * 16, 16)], zsem)
    for i in range(RPT // 16):
        pltpu.make_async_copy(zbuf, acc_sh.at[pl.ds(r0, 16)], zsem).wait()
    plsc.subcore_barrier()

    @pl.loop(0, KM, step=8)
    def _edges(j):
        for b in range(8):
            br = b % 4
            gwait(br)
            scat(b, br)
            swait(br)

            @pl.when(j + b + 8 < K)
            def _():
                iload(j + b + 8, b)

            @pl.when(j + b + 4 < K)
            def _():
                iwait((b + 4) % 8)
                gather((b + 4) % 8, br)

    for jj in range(KM, K):
        br = jj % 4
        gwait(br)
        scat(jj % 8, br)
        swait(br)
        if jj + 4 < K:
            iwait((jj + 4) % 8)
            gather((jj + 4) % 8, br)

    plsc.subcore_barrier()
    pltpu.sync_copy(acc_sh.at[pl.ds(r0, RPT)],
                    out_hbm.at[cid, pl.ds(r0, RPT)])


_BM = 1000


def _g_body(x_ref, w_ref, pt_ref, g_ref):
    d = pt_ref[:, 0] + pt_ref[:, 1] + 1.0
    dinv = lax.rsqrt(d)
    h = jnp.dot(x_ref[:, :], w_ref[:, :], preferred_element_type=jnp.float32)
    g_ref[:, :] = h * dinv[:, None]


_g_call = pl.pallas_call(
    _g_body,
    grid=(N // _BM,),
    in_specs=[
        pl.BlockSpec((_BM, D), lambda i: (i, 0)),
        pl.BlockSpec((D, D), lambda i: (0, 0)),
        pl.BlockSpec((_BM, 2), lambda i: (i, 0)),
    ],
    out_specs=pl.BlockSpec((_BM, D), lambda i: (i, 0)),
    out_shape=jax.ShapeDtypeStruct((N, D), jnp.float32),
)


def _fin_body(acc_ref, g_ref, pt_ref, b_ref, o_ref):
    d = pt_ref[:, 0] + pt_ref[:, 1] + 1.0
    dinv = lax.rsqrt(d)
    s = acc_ref[0] + acc_ref[1] + g_ref[:, :]
    o_ref[:, :] = s * dinv[:, None] + b_ref[0]


_fin_call = pl.pallas_call(
    _fin_body,
    grid=(N // _BM,),
    in_specs=[
        pl.BlockSpec((NSC, _BM, D), lambda i: (0, i, 0)),
        pl.BlockSpec((_BM, D), lambda i: (i, 0)),
        pl.BlockSpec((_BM, 2), lambda i: (i, 0)),
        pl.BlockSpec((1, D), lambda i: (0, 0)),
    ],
    out_specs=pl.BlockSpec((_BM, D), lambda i: (i, 0)),
    out_shape=jax.ShapeDtypeStruct((N, D), jnp.float32),
)


def kernel(x, edge_index, W, b):
    e_flat = edge_index.reshape(2 * E)
    degp = _deg_kernel(e_flat)
    pt = degp.T
    g = _g_call(x, W, pt)
    accs = _agg_kernel(g, e_flat)
    out = _fin_call(accs, g, pt, b.reshape(1, D))
    return out


# speedup vs baseline: 1.2515x; 1.0502x over previous
import functools

import jax
import jax.numpy as jnp
from jax import lax
from jax.experimental import pallas as pl
from jax.experimental.pallas import tpu as pltpu
from jax.experimental.pallas import tpu_sc as plsc

N = 10000
D = 128
E = 320000
NP = 10240
C = 80
NSC = 2
NSUB = 16
NW = NSC * NSUB
K = 125
EPW = K * C
RPT = NP // NSUB
KM = 120

_sc_mesh = plsc.VectorSubcoreMesh(core_axis_name="c", subcore_axis_name="s")


@functools.partial(
    pl.kernel,
    out_type=jax.ShapeDtypeStruct((NSC, NP), jnp.float32),
    mesh=_sc_mesh,
    scratch_types=[
        pltpu.VMEM((K, C), jnp.int32),
        pltpu.VMEM((C,), jnp.float32),
        pltpu.VMEM((RPT,), jnp.float32),
        pltpu.VMEM_SHARED((NP,), jnp.float32),
        pltpu.SemaphoreType.DMA,
        pltpu.SemaphoreType.DMA,
    ],
)
def _deg_kernel(edge_hbm, out_hbm, didx_v, ones_v, zrow_v, deg_sh, sem, isem):
    cid = lax.axis_index("c")
    sid = lax.axis_index("s")
    wid = sid * NSC + cid
    base = wid * EPW

    @pl.loop(0, K)
    def _il(j):
        pltpu.async_copy(edge_hbm.at[pl.ds(E + base + j * C, C)],
                         didx_v.at[j], isem)

    for i in range(C // 16):
        ones_v[pl.ds(i * 16, 16)] = jnp.ones((16,), jnp.float32)
    for i in range(RPT // 16):
        zrow_v[pl.ds(i * 16, 16)] = jnp.zeros((16,), jnp.float32)
    r0 = sid * RPT
    pltpu.sync_copy(zrow_v, deg_sh.at[pl.ds(r0, RPT)])

    @pl.loop(0, K)
    def _ilw(j):
        pltpu.make_async_copy(edge_hbm.at[pl.ds(E + base, C)],
                              didx_v.at[0], isem).wait()

    plsc.subcore_barrier()

    @pl.loop(0, K)
    def _fire(j):
        pltpu.async_copy(ones_v, deg_sh.at[didx_v.at[j]], sem, add=True)

    @pl.loop(0, K)
    def _drain(j):
        pltpu.make_async_copy(ones_v, deg_sh.at[didx_v.at[0]], sem).wait()

    plsc.subcore_barrier()
    pltpu.sync_copy(deg_sh.at[pl.ds(r0, RPT)], out_hbm.at[cid, pl.ds(r0, RPT)])


@functools.partial(
    pl.kernel,
    out_type=jax.ShapeDtypeStruct((NSC, NP, D), jnp.float32),
    mesh=_sc_mesh,
    scratch_types=[
        [pltpu.VMEM((1, C), jnp.int32) for _ in range(8)],
        [pltpu.VMEM((1, C), jnp.int32) for _ in range(8)],
        [pltpu.VMEM((C, D), jnp.float32) for _ in range(4)],
        pltpu.VMEM((16, D), jnp.float32),
        pltpu.VMEM_SHARED((NP, D), jnp.float32),
        [pltpu.SemaphoreType.DMA for _ in range(8)],
        [pltpu.SemaphoreType.DMA for _ in range(4)],
        [pltpu.SemaphoreType.DMA for _ in range(4)],
        pltpu.SemaphoreType.DMA,
    ],
)
def _agg_kernel(g_hbm, edge_hbm, out_hbm,
                sidx, didx, rows, zbuf, acc_sh, isem, gsem, ssem, zsem):
    cid = lax.axis_index("c")
    sid = lax.axis_index("s")
    wid = sid * NSC + cid
    r0 = sid * RPT
    base = wid * EPW

    def iload(j, bi):
        pltpu.async_copy(edge_hbm.at[pl.ds(base + j * C, C)],
                         sidx[bi].at[0], isem[bi])
        pltpu.async_copy(edge_hbm.at[pl.ds(E + base + j * C, C)],
                         didx[bi].at[0], isem[bi])

    def iwait(bi):
        pltpu.make_async_copy(edge_hbm.at[pl.ds(base, C)],
                              sidx[bi].at[0], isem[bi]).wait()
        pltpu.make_async_copy(edge_hbm.at[pl.ds(E + base, C)],
                              didx[bi].at[0], isem[bi]).wait()

    def gather(bi, br):
        pltpu.async_copy(g_hbm.at[sidx[bi].at[0]], rows[br], gsem[br])

    def gwait(br):
        pltpu.make_async_copy(g_hbm.at[sidx[0].at[0]], rows[br],
                              gsem[br]).wait()

    def scat(bi, br):
        pltpu.async_copy(rows[br], acc_sh.at[didx[bi].at[0]], ssem[br],
                         add=True)

    def swait(br):
        pltpu.make_async_copy(rows[br], acc_sh.at[didx[0].at[0]],
                              ssem[br]).wait()

    for j in range(8):
        iload(j, j)
    for b in range(4):
        iwait(b)
        gather(b, b)

    for i in range(16):
        for k in range(D // 16):
            zbuf[i, pl.ds(k * 16, 16)] = jnp.zeros((16,), jnp.float32)
    for i in range(RPT // 16):
        pltpu.async_copy(zbuf, acc_sh.at[pl.ds(r0 + i * 16, 16)], zsem)
    for i in range(RPT // 16):
        pltpu.make_async_copy(zbuf, acc_sh.at[pl.ds(r0, 16)], zsem).wait()
    plsc.subcore_barrier()

    @pl.loop(0, KM, step=8)
    def _edges(j):
        for b in range(8):
            br = b % 4
            gwait(br)
            scat(b, br)
            swait(br)

            @pl.when(j + b + 8 < K)
            def _():
                iload(j + b + 8, b)

            @pl.when(j + b + 4 < K)
            def _():
                iwait((b + 4) % 8)
                gather((b + 4) % 8, br)

    for jj in range(KM, K):
        br = jj % 4
        gwait(br)
        scat(jj % 8, br)
        swait(br)
        if jj + 4 < K:
            iwait((jj + 4) % 8)
            gather((jj + 4) % 8, br)

    plsc.subcore_barrier()
    pltpu.sync_copy(acc_sh.at[pl.ds(r0, RPT)],
                    out_hbm.at[cid, pl.ds(r0, RPT)])


_BM = 1024


def _g_body(x_ref, w_ref, dg_ref, g_ref):
    d = dg_ref[0] + dg_ref[1] + 1.0
    dinv = lax.rsqrt(d)
    h = jnp.dot(x_ref[:, :], w_ref[:, :], preferred_element_type=jnp.float32)
    g_ref[:, :] = h * dinv[:, None]


_g_call = pl.pallas_call(
    _g_body,
    grid=(NP // _BM,),
    in_specs=[
        pl.BlockSpec((_BM, D), lambda i: (i, 0)),
        pl.BlockSpec((D, D), lambda i: (0, 0)),
        pl.BlockSpec((NSC, _BM), lambda i: (0, i)),
    ],
    out_specs=pl.BlockSpec((_BM, D), lambda i: (i, 0)),
    out_shape=jax.ShapeDtypeStruct((NP, D), jnp.float32),
)


def _fin_body(acc_ref, g_ref, dg_ref, b_ref, o_ref):
    d = dg_ref[0] + dg_ref[1] + 1.0
    dinv = lax.rsqrt(d)
    s = acc_ref[0] + acc_ref[1] + g_ref[:, :]
    o_ref[:, :] = s * dinv[:, None] + b_ref[0]


_fin_call = pl.pallas_call(
    _fin_body,
    grid=(NP // _BM,),
    in_specs=[
        pl.BlockSpec((NSC, _BM, D), lambda i: (0, i, 0)),
        pl.BlockSpec((_BM, D), lambda i: (i, 0)),
        pl.BlockSpec((NSC, _BM), lambda i: (0, i)),
        pl.BlockSpec((1, D), lambda i: (0, 0)),
    ],
    out_specs=pl.BlockSpec((_BM, D), lambda i: (i, 0)),
    out_shape=jax.ShapeDtypeStruct((N, D), jnp.float32),
)


def kernel(x, edge_index, W, b):
    e_flat = edge_index.reshape(2 * E)
    degp = _deg_kernel(e_flat)
    g = _g_call(x, W, degp)
    accs = _agg_kernel(g, e_flat)
    out = _fin_call(accs, g, degp, b.reshape(1, D))
    return out
